# baseline (speedup 1.0000x reference)
import functools

import jax
import jax.numpy as jnp
from jax import lax
from jax.experimental import pallas as pl
from jax.experimental.pallas import tpu as pltpu
from jax.experimental.pallas import tpu_sc as plsc

VOCAB = 1_000_000
EMBED_DIM = 64
BATCH = 16384
CHUNK = 128


@functools.lru_cache(maxsize=None)
def _build_gather():
    info = plsc.get_sparse_core_info()
    nw = info.num_cores * info.num_subcores
    b_per_w = BATCH // nw
    n_chunks = b_per_w // CHUNK
    mesh = plsc.VectorSubcoreMesh(core_axis_name="c", subcore_axis_name="s")

    @functools.partial(
        pl.kernel,
        out_type=jax.ShapeDtypeStruct((BATCH, EMBED_DIM), jnp.float32),
        mesh=mesh,
        scratch_types=[
            pltpu.VMEM((n_chunks, CHUNK), jnp.int32),
            pltpu.VMEM((b_per_w, EMBED_DIM), jnp.float32),
            pltpu.SemaphoreType.DMA,
        ],
        compiler_params=pltpu.CompilerParams(use_tc_tiling_on_sc=False),
    )
    def gather_kernel(idx_hbm, table_hbm, out_hbm, idx_v, rows_v, sem):
        wid = lax.axis_index("s") * info.num_cores + lax.axis_index("c")
        base = wid * b_per_w
        pltpu.sync_copy(idx_hbm.at[pl.ds(wid * n_chunks, n_chunks)], idx_v)
        copies = [
            pltpu.async_copy(
                table_hbm.at[idx_v.at[j]],
                rows_v.at[pl.ds(j * CHUNK, CHUNK)],
                sem,
            )
            for j in range(n_chunks)
        ]
        for c in copies:
            c.wait()
        pltpu.sync_copy(rows_v, out_hbm.at[pl.ds(base, b_per_w)])

    return gather_kernel


def kernel(indices, embed_cache):
    idx2d = indices.astype(jnp.int32).reshape(BATCH // CHUNK, CHUNK)
    return _build_gather()(idx2d, embed_cache)


# speedup vs baseline: 1.0333x; 1.0333x over previous
import functools

import jax
import jax.numpy as jnp
from jax import lax
from jax.experimental import pallas as pl
from jax.experimental.pallas import tpu as pltpu
from jax.experimental.pallas import tpu_sc as plsc

VOCAB = 1_000_000
EMBED_DIM = 64
BATCH = 16384
IDX_COLS = 128
GROUP = 16


@functools.lru_cache(maxsize=None)
def _build_gather():
    info = plsc.get_sparse_core_info()
    nw = info.num_cores * info.num_subcores
    b_per_w = BATCH // nw
    idx_rows = b_per_w // IDX_COLS
    n_groups = b_per_w // GROUP
    mesh = plsc.VectorSubcoreMesh(core_axis_name="c", subcore_axis_name="s")

    @functools.partial(
        pl.kernel,
        out_type=jax.ShapeDtypeStruct((BATCH, EMBED_DIM), jnp.float32),
        mesh=mesh,
        scratch_types=[
            pltpu.VMEM((idx_rows, IDX_COLS), jnp.int32),
            pltpu.SemaphoreType.DMA,
        ],
    )
    def gather_kernel(idx_hbm, table_hbm, out_hbm, idx_v, sem):
        wid = lax.axis_index("s") * info.num_cores + lax.axis_index("c")
        base = wid * b_per_w
        pltpu.sync_copy(idx_hbm.at[pl.ds(wid * idx_rows, idx_rows)], idx_v)

        def group_body(g, _):
            i0 = g * GROUP
            idx_vec = idx_v[i0 // IDX_COLS, pl.ds(i0 % IDX_COLS, GROUP)]
            copies = []
            for kk in range(GROUP):
                r = idx_vec[kk]
                copies.append(
                    pltpu.async_copy(
                        table_hbm.at[pl.ds(r, 1)],
                        out_hbm.at[pl.ds(base + i0 + kk, 1)],
                        sem,
                    )
                )
            for c in copies:
                c.wait()
            return 0

        lax.fori_loop(0, n_groups, group_body, 0)

    return gather_kernel


def kernel(indices, embed_cache):
    idx2d = indices.astype(jnp.int32).reshape(BATCH // IDX_COLS, IDX_COLS)
    return _build_gather()(idx2d, embed_cache)


# speedup vs baseline: 1.0348x; 1.0015x over previous
import functools

import jax
import jax.numpy as jnp
from jax import lax
from jax.experimental import pallas as pl
from jax.experimental.pallas import tpu as pltpu
from jax.experimental.pallas import tpu_sc as plsc

VOCAB = 1_000_000
EMBED_DIM = 64
BATCH = 16384
IDX_COLS = 128
GROUP = 64


@functools.lru_cache(maxsize=None)
def _build_gather():
    info = plsc.get_sparse_core_info()
    nw = info.num_cores * info.num_subcores
    b_per_w = BATCH // nw
    idx_rows = b_per_w // IDX_COLS
    n_groups = b_per_w // GROUP
    mesh = plsc.VectorSubcoreMesh(core_axis_name="c", subcore_axis_name="s")

    @functools.partial(
        pl.kernel,
        out_type=jax.ShapeDtypeStruct((BATCH, EMBED_DIM), jnp.float32),
        mesh=mesh,
        scratch_types=[
            pltpu.VMEM((idx_rows, IDX_COLS), jnp.int32),
            pltpu.SemaphoreType.DMA,
        ],
        compiler_params=pltpu.CompilerParams(use_tc_tiling_on_sc=True),
    )
    def gather_kernel(idx_hbm, table_hbm, out_hbm, idx_v, sem):
        wid = lax.axis_index("s") * info.num_cores + lax.axis_index("c")
        base = wid * b_per_w
        pltpu.sync_copy(idx_hbm.at[pl.ds(wid * idx_rows, idx_rows)], idx_v)

        def group_body(g, _):
            i0 = g * GROUP
            for half in range(GROUP // 16):
                j0 = i0 + half * 16
                idx_vec = idx_v[j0 // IDX_COLS, pl.ds(j0 % IDX_COLS, 16)]
                for kk in range(16):
                    r = idx_vec[kk]
                    pltpu.async_copy(
                        table_hbm.at[pl.ds(r, 1)],
                        out_hbm.at[pl.ds(base + j0 + kk, 1)],
                        sem,
                    )
            for kk in range(GROUP):
                pltpu.make_async_copy(
                    table_hbm.at[pl.ds(0, 1)],
                    out_hbm.at[pl.ds(base, 1)],
                    sem,
                ).wait()
            return 0

        lax.fori_loop(0, n_groups, group_body, 0)

    return gather_kernel


def kernel(indices, embed_cache):
    idx2d = indices.astype(jnp.int32).reshape(BATCH // IDX_COLS, IDX_COLS)
    return _build_gather()(idx2d, embed_cache)


# speedup vs baseline: 2.2079x; 2.1336x over previous
import functools

import jax
import jax.numpy as jnp
from jax import lax
from jax.experimental import pallas as pl
from jax.experimental.pallas import tpu as pltpu
from jax.experimental.pallas import tpu_sc as plsc

VOCAB = 1_000_000
EMBED_DIM = 64
BATCH = 16384
GRAN = 16
IDX_COLS = 128
VMAIN = 999_936
GPR = VMAIN // GRAN
SEG = 7936
N_SEGS = VMAIN // SEG


@functools.lru_cache(maxsize=None)
def _build_detile():
    info = plsc.get_sparse_core_info()
    nw = info.num_cores * info.num_subcores
    n_items = (EMBED_DIM // 8) * N_SEGS
    max_iters = (n_items + nw - 1) // nw
    mesh = plsc.VectorSubcoreMesh(core_axis_name="c", subcore_axis_name="s")

    @functools.partial(
        pl.kernel,
        out_type=jax.ShapeDtypeStruct((VMAIN * EMBED_DIM,), jnp.float32),
        mesh=mesh,
        scratch_types=[
            pltpu.VMEM((8, SEG), jnp.float32),
            pltpu.VMEM((8, SEG), jnp.float32),
            pltpu.SemaphoreType.DMA,
            pltpu.SemaphoreType.DMA,
            pltpu.SemaphoreType.DMA,
            pltpu.SemaphoreType.DMA,
        ],
        compiler_params=pltpu.CompilerParams(use_tc_tiling_on_sc=True,
                                             needs_layout_passes=False),
    )
    def detile_kernel(table_hbm, flat_hbm, buf_a, buf_b,
                      lsem_a, lsem_b, ssem_a, ssem_b):
        wid = lax.axis_index("s") * info.num_cores + lax.axis_index("c")
        bufs = (buf_a, buf_b)
        lsems, ssems = (lsem_a, lsem_b), (ssem_a, ssem_b)

        def valid(i):
            return wid + i * nw < n_items

        def band_seg(i):
            item = wid + i * nw
            return item // N_SEGS, item % N_SEGS

        def start_load(i):
            band, s = band_seg(i)
            p = i % 2

            @pl.when(valid(i))
            def _():
                pltpu.make_async_copy(
                    table_hbm.at[pl.ds(band * 8, 8), pl.ds(s * SEG, SEG)],
                    bufs[p], lsems[p]).start()

        def wait_load(i):
            p = i % 2

            @pl.when(valid(i))
            def _():
                pltpu.make_async_copy(
                    table_hbm.at[pl.ds(0, 8), pl.ds(0, SEG)],
                    bufs[p], lsems[p]).wait()

        def start_stores(i):
            band, s = band_seg(i)
            p = i % 2

            @pl.when(valid(i))
            def _():
                for r in range(8):
                    pltpu.make_async_copy(
                        bufs[p].at[r],
                        flat_hbm.at[pl.ds((band * 8 + r) * VMAIN + s * SEG,
                                          SEG)],
                        ssems[p]).start()

        def wait_stores(i):
            p = i % 2

            @pl.when(valid(i))
            def _():
                for r in range(8):
                    pltpu.make_async_copy(
                        bufs[p].at[r],
                        flat_hbm.at[pl.ds(0, SEG)],
                        ssems[p]).wait()

        start_load(0)
        for i in range(max_iters):
            wait_load(i)
            start_stores(i)
            if i >= 1:
                wait_stores(i - 1)
            if i + 1 < max_iters:
                start_load(i + 1)
        wait_stores(max_iters - 1)

    return detile_kernel


@functools.lru_cache(maxsize=None)
def _build_gather():
    info = plsc.get_sparse_core_info()
    nw = info.num_cores * info.num_subcores
    b_per_w = BATCH // nw
    idx_rows = b_per_w // IDX_COLS
    mesh = plsc.VectorSubcoreMesh(core_axis_name="c", subcore_axis_name="s")

    @functools.partial(
        pl.kernel,
        out_type=jax.ShapeDtypeStruct((EMBED_DIM, BATCH), jnp.float32),
        mesh=mesh,
        scratch_types=[
            pltpu.VMEM((idx_rows, IDX_COLS), jnp.int32),
            pltpu.VMEM((idx_rows, IDX_COLS), jnp.int32),
            pltpu.VMEM((idx_rows, IDX_COLS), jnp.int32),
            pltpu.VMEM((idx_rows, IDX_COLS), jnp.int32),
            pltpu.VMEM((2, 1, b_per_w), jnp.int32),
            pltpu.VMEM((EMBED_DIM * EMBED_DIM // GRAN, GRAN),
                       jnp.float32),
            pltpu.VMEM((2, b_per_w, GRAN), jnp.float32),
            pltpu.VMEM((2, 1, b_per_w), jnp.float32),
            pltpu.SemaphoreType.DMA,
            pltpu.SemaphoreType.DMA,
            pltpu.SemaphoreType.DMA,
            pltpu.SemaphoreType.DMA,
        ],
        compiler_params=pltpu.CompilerParams(use_tc_tiling_on_sc=False,
                                             needs_layout_passes=False),
    )
    def gather_kernel(g_hbm, s_hbm, gs_hbm, sel_hbm, small_hbm, gran_hbm,
                      out_hbm, g_v, s_v, gs_v, sel_v, gc_v, small_v,
                      gath_v, comp_v, gsem_a, gsem_b, osem_a, osem_b):
        wid = lax.axis_index("s") * info.num_cores + lax.axis_index("c")
        pltpu.sync_copy(g_hbm.at[pl.ds(wid * idx_rows, idx_rows)], g_v)
        pltpu.sync_copy(s_hbm.at[pl.ds(wid * idx_rows, idx_rows)], s_v)
        pltpu.sync_copy(gs_hbm.at[pl.ds(wid * idx_rows, idx_rows)], gs_v)
        pltpu.sync_copy(sel_hbm.at[pl.ds(wid * idx_rows, idx_rows)], sel_v)
        pltpu.sync_copy(small_hbm, small_v)
        iota16 = lax.iota(jnp.int32, 16)
        gsems = (gsem_a, gsem_b)
        osems = (osem_a, osem_b)

        def fire(c_row, p):
            base = c_row * GPR
            for r in range(idx_rows):
                for q in range(IDX_COLS // 16):
                    gc_v[p, 0, pl.ds(r * IDX_COLS + q * 16, 16)] = (
                        g_v[r, pl.ds(q * 16, 16)] + base)
            pltpu.make_async_copy(
                gran_hbm.at[gc_v.at[p, 0]],
                gath_v.at[p],
                gsems[p]).start()

        def drain(p):
            pltpu.make_async_copy(
                gran_hbm.at[gc_v.at[p, 0]],
                gath_v.at[p],
                gsems[p]).wait()

        def compact_and_store(c_row, p):
            small_base = c_row * (EMBED_DIM // GRAN)
            for k in range(b_per_w // 16):
                rr, cc = (k * 16) // IDX_COLS, (k * 16) % IDX_COLS
                s_vec = s_v[rr, pl.ds(cc, 16)]
                rows_k = iota16 + (k * 16)
                main16 = plsc.load_gather(gath_v.at[p], [rows_k, s_vec])
                srow = gs_v[rr, pl.ds(cc, 16)] + small_base
                small16 = plsc.load_gather(small_v, [srow, s_vec])
                sel = sel_v[rr, pl.ds(cc, 16)]
                comp_v[p, 0, pl.ds(k * 16, 16)] = jnp.where(
                    sel > 0, small16, main16)
            pltpu.make_async_copy(
                comp_v.at[p],
                out_hbm.at[pl.ds(c_row, 1), pl.ds(wid * b_per_w, b_per_w)],
                osems[p]).start()

        def wait_out(p):
            pltpu.make_async_copy(
                comp_v.at[p],
                out_hbm.at[pl.ds(0, 1), pl.ds(0, b_per_w)],
                osems[p]).wait()

        fire(0, 0)

        def pair(p2, _):
            for par in range(2):
                c_row = p2 * 2 + par

                @pl.when(c_row + 1 < EMBED_DIM)
                def _():
                    fire(c_row + 1, 1 - par)

                drain(par)

                @pl.when(c_row >= 2)
                def _():
                    wait_out(par)

                compact_and_store(c_row, par)
            return 0

        lax.fori_loop(0, EMBED_DIM // 2, pair, 0)
        wait_out(0)
        wait_out(1)

    return gather_kernel


def kernel(indices, embed_cache):
    idx = indices.astype(jnp.int32)
    shape2d = (BATCH // IDX_COLS, IDX_COLS)
    g2d = (jnp.minimum(idx, VMAIN - 1) // GRAN).reshape(shape2d)
    s2d = (idx % GRAN).reshape(shape2d)
    gs2d = (jnp.clip(idx - VMAIN, 0, VOCAB - VMAIN - 1) // GRAN).reshape(
        shape2d)
    sel2d = (idx >= VMAIN).astype(jnp.int32).reshape(shape2d)
    small = embed_cache.T[:, VMAIN:].reshape(
        EMBED_DIM * (VOCAB - VMAIN) // GRAN, GRAN)
    flat = _build_detile()(embed_cache.T)
    out_t = _build_gather()(
        g2d, s2d, gs2d, sel2d, small,
        flat.reshape(VMAIN * EMBED_DIM // GRAN, GRAN))
    return out_t.T


# speedup vs baseline: 2.3445x; 1.0619x over previous
import functools

import jax
import jax.numpy as jnp
from jax import lax
from jax.experimental import pallas as pl
from jax.experimental.pallas import tpu as pltpu
from jax.experimental.pallas import tpu_sc as plsc

VOCAB = 1_000_000
EMBED_DIM = 64
BATCH = 16384
GRAN = 16
IDX_COLS = 128
VMAIN = 999_936
GPR = VMAIN // GRAN
SEG = 7936
N_SEGS = VMAIN // SEG


@functools.lru_cache(maxsize=None)
def _build_detile():
    info = plsc.get_sparse_core_info()
    nw = info.num_cores * info.num_subcores
    n_items = (EMBED_DIM // 8) * N_SEGS
    max_iters = (n_items + nw - 1) // nw
    mesh = plsc.VectorSubcoreMesh(core_axis_name="c", subcore_axis_name="s")

    @functools.partial(
        pl.kernel,
        out_type=jax.ShapeDtypeStruct((VMAIN * EMBED_DIM,), jnp.float32),
        mesh=mesh,
        scratch_types=[
            pltpu.VMEM((8, SEG), jnp.float32),
            pltpu.VMEM((8, SEG), jnp.float32),
            pltpu.SemaphoreType.DMA,
            pltpu.SemaphoreType.DMA,
            pltpu.SemaphoreType.DMA,
            pltpu.SemaphoreType.DMA,
        ],
        compiler_params=pltpu.CompilerParams(use_tc_tiling_on_sc=True,
                                             needs_layout_passes=False),
    )
    def detile_kernel(table_hbm, flat_hbm, buf_a, buf_b,
                      lsem_a, lsem_b, ssem_a, ssem_b):
        wid = lax.axis_index("s") * info.num_cores + lax.axis_index("c")
        bufs = (buf_a, buf_b)
        lsems, ssems = (lsem_a, lsem_b), (ssem_a, ssem_b)

        def valid(i):
            return wid + i * nw < n_items

        def band_seg(i):
            item = wid + i * nw
            return item // N_SEGS, item % N_SEGS

        def start_load(i):
            band, s = band_seg(i)
            p = i % 2

            @pl.when(valid(i))
            def _():
                pltpu.make_async_copy(
                    table_hbm.at[pl.ds(band * 8, 8), pl.ds(s * SEG, SEG)],
                    bufs[p], lsems[p]).start()

        def wait_load(i):
            p = i % 2

            @pl.when(valid(i))
            def _():
                pltpu.make_async_copy(
                    table_hbm.at[pl.ds(0, 8), pl.ds(0, SEG)],
                    bufs[p], lsems[p]).wait()

        def do_stores(i):
            band, s = band_seg(i)
            p = i % 2

            @pl.when(valid(i))
            def _():
                for r in range(8):
                    pltpu.sync_copy(
                        bufs[p].at[r],
                        flat_hbm.at[pl.ds((band * 8 + r) * VMAIN + s * SEG,
                                          SEG)])

        start_load(0)
        for i in range(max_iters):
            wait_load(i)
            if i + 1 < max_iters:
                start_load(i + 1)
            do_stores(i)

    return detile_kernel


@functools.lru_cache(maxsize=None)
def _build_gather():
    info = plsc.get_sparse_core_info()
    nw = info.num_cores * info.num_subcores
    b_per_w = BATCH // nw
    idx_rows = b_per_w // IDX_COLS
    mesh = plsc.VectorSubcoreMesh(core_axis_name="c", subcore_axis_name="s")

    @functools.partial(
        pl.kernel,
        out_type=jax.ShapeDtypeStruct((EMBED_DIM, BATCH), jnp.float32),
        mesh=mesh,
        scratch_types=[
            pltpu.VMEM((idx_rows, IDX_COLS), jnp.int32),
            pltpu.VMEM((idx_rows, IDX_COLS), jnp.int32),
            pltpu.VMEM((idx_rows, IDX_COLS), jnp.int32),
            pltpu.VMEM((idx_rows, IDX_COLS), jnp.int32),
            pltpu.VMEM((2, 1, b_per_w), jnp.int32),
            pltpu.VMEM((EMBED_DIM * EMBED_DIM // GRAN, GRAN),
                       jnp.float32),
            pltpu.VMEM((2, b_per_w, GRAN), jnp.float32),
            pltpu.VMEM((2, 1, b_per_w), jnp.float32),
            pltpu.SemaphoreType.DMA,
            pltpu.SemaphoreType.DMA,
            pltpu.SemaphoreType.DMA,
            pltpu.SemaphoreType.DMA,
        ],
        compiler_params=pltpu.CompilerParams(use_tc_tiling_on_sc=False,
                                             needs_layout_passes=False),
    )
    def gather_kernel(g_hbm, s_hbm, gs_hbm, sel_hbm, small_hbm, gran_hbm,
                      out_hbm, g_v, s_v, gs_v, sel_v, gc_v, small_v,
                      gath_v, comp_v, gsem_a, gsem_b, osem_a, osem_b):
        wid = lax.axis_index("s") * info.num_cores + lax.axis_index("c")
        pltpu.sync_copy(g_hbm.at[pl.ds(wid * idx_rows, idx_rows)], g_v)
        pltpu.sync_copy(s_hbm.at[pl.ds(wid * idx_rows, idx_rows)], s_v)
        pltpu.sync_copy(gs_hbm.at[pl.ds(wid * idx_rows, idx_rows)], gs_v)
        pltpu.sync_copy(sel_hbm.at[pl.ds(wid * idx_rows, idx_rows)], sel_v)
        pltpu.sync_copy(small_hbm, small_v)
        iota16 = lax.iota(jnp.int32, 16)
        gsems = (gsem_a, gsem_b)
        osems = (osem_a, osem_b)

        def fire(c_row, p):
            base = c_row * GPR
            for r in range(idx_rows):
                for q in range(IDX_COLS // 16):
                    gc_v[p, 0, pl.ds(r * IDX_COLS + q * 16, 16)] = (
                        g_v[r, pl.ds(q * 16, 16)] + base)
            pltpu.make_async_copy(
                gran_hbm.at[gc_v.at[p, 0]],
                gath_v.at[p],
                gsems[p]).start()

        def drain(p):
            pltpu.make_async_copy(
                gran_hbm.at[gc_v.at[p, 0]],
                gath_v.at[p],
                gsems[p]).wait()

        def compact_and_store(c_row, p):
            small_base = c_row * (EMBED_DIM // GRAN)
            for k in range(b_per_w // 16):
                rr, cc = (k * 16) // IDX_COLS, (k * 16) % IDX_COLS
                s_vec = s_v[rr, pl.ds(cc, 16)]
                rows_k = iota16 + (k * 16)
                main16 = plsc.load_gather(gath_v.at[p], [rows_k, s_vec])
                srow = gs_v[rr, pl.ds(cc, 16)] + small_base
                small16 = plsc.load_gather(small_v, [srow, s_vec])
                sel = sel_v[rr, pl.ds(cc, 16)]
                comp_v[p, 0, pl.ds(k * 16, 16)] = jnp.where(
                    sel > 0, small16, main16)
            pltpu.make_async_copy(
                comp_v.at[p],
                out_hbm.at[pl.ds(c_row, 1), pl.ds(wid * b_per_w, b_per_w)],
                osems[p]).start()

        def wait_out(p):
            pltpu.make_async_copy(
                comp_v.at[p],
                out_hbm.at[pl.ds(0, 1), pl.ds(0, b_per_w)],
                osems[p]).wait()

        fire(0, 0)

        def pair(p2, _):
            for par in range(2):
                c_row = p2 * 2 + par

                @pl.when(c_row + 1 < EMBED_DIM)
                def _():
                    fire(c_row + 1, 1 - par)

                drain(par)

                @pl.when(c_row >= 2)
                def _():
                    wait_out(par)

                compact_and_store(c_row, par)
            return 0

        lax.fori_loop(0, EMBED_DIM // 2, pair, 0)
        wait_out(0)
        wait_out(1)

    return gather_kernel


def kernel(indices, embed_cache):
    idx = indices.astype(jnp.int32)
    shape2d = (BATCH // IDX_COLS, IDX_COLS)
    g2d = (jnp.minimum(idx, VMAIN - 1) // GRAN).reshape(shape2d)
    s2d = (idx % GRAN).reshape(shape2d)
    gs2d = (jnp.clip(idx - VMAIN, 0, VOCAB - VMAIN - 1) // GRAN).reshape(
        shape2d)
    sel2d = (idx >= VMAIN).astype(jnp.int32).reshape(shape2d)
    small = embed_cache.T[:, VMAIN:].reshape(
        EMBED_DIM * (VOCAB - VMAIN) // GRAN, GRAN)
    flat = _build_detile()(embed_cache.T)
    out_t = _build_gather()(
        g2d, s2d, gs2d, sel2d, small,
        flat.reshape(VMAIN * EMBED_DIM // GRAN, GRAN))
    return out_t.T


# speedup vs baseline: 2.3671x; 1.0096x over previous
import functools

import jax
import jax.numpy as jnp
from jax import lax
from jax.experimental import pallas as pl
from jax.experimental.pallas import tpu as pltpu
from jax.experimental.pallas import tpu_sc as plsc

VOCAB = 1_000_000
EMBED_DIM = 64
BATCH = 16384
GRAN = 16
IDX_COLS = 128
VMAIN = 999_936
GPR = VMAIN // GRAN
SEG = 7936
N_SEGS = VMAIN // SEG


@functools.lru_cache(maxsize=None)
def _build_detile():
    info = plsc.get_sparse_core_info()
    nw = info.num_cores * info.num_subcores
    n_items = (EMBED_DIM // 8) * N_SEGS
    max_iters = (n_items + nw - 1) // nw
    mesh = plsc.VectorSubcoreMesh(core_axis_name="c", subcore_axis_name="s")

    @functools.partial(
        pl.kernel,
        out_type=jax.ShapeDtypeStruct((VMAIN * EMBED_DIM,), jnp.float32),
        mesh=mesh,
        scratch_types=[
            pltpu.VMEM((8, SEG), jnp.float32),
            pltpu.VMEM((8, SEG), jnp.float32),
            pltpu.SemaphoreType.DMA,
            pltpu.SemaphoreType.DMA,
            pltpu.SemaphoreType.DMA,
            pltpu.SemaphoreType.DMA,
        ],
        compiler_params=pltpu.CompilerParams(use_tc_tiling_on_sc=True,
                                             needs_layout_passes=False),
    )
    def detile_kernel(table_hbm, flat_hbm, buf_a, buf_b,
                      lsem_a, lsem_b, ssem_a, ssem_b):
        wid = lax.axis_index("s") * info.num_cores + lax.axis_index("c")
        bufs = (buf_a, buf_b)
        lsems, ssems = (lsem_a, lsem_b), (ssem_a, ssem_b)

        def valid(i):
            return wid + i * nw < n_items

        def band_seg(i):
            item = wid + i * nw
            return item // N_SEGS, item % N_SEGS

        def start_load(i):
            band, s = band_seg(i)
            p = i % 2

            @pl.when(valid(i))
            def _():
                pltpu.make_async_copy(
                    table_hbm.at[pl.ds(band * 8, 8), pl.ds(s * SEG, SEG)],
                    bufs[p], lsems[p]).start()

        def wait_load(i):
            p = i % 2

            @pl.when(valid(i))
            def _():
                pltpu.make_async_copy(
                    table_hbm.at[pl.ds(0, 8), pl.ds(0, SEG)],
                    bufs[p], lsems[p]).wait()

        def start_stores(i):
            band, s = band_seg(i)
            p = i % 2

            @pl.when(valid(i))
            def _():
                for r in range(8):
                    pltpu.make_async_copy(
                        bufs[p].at[r],
                        flat_hbm.at[pl.ds((band * 8 + r) * VMAIN + s * SEG,
                                          SEG)],
                        ssems[p]).start()

        def wait_stores(i):
            p = i % 2

            @pl.when(valid(i))
            def _():
                pltpu.make_async_copy(
                    table_hbm.at[pl.ds(0, 8), pl.ds(0, SEG)],
                    bufs[p], ssems[p]).wait()

        start_load(0)
        for i in range(max_iters):
            wait_load(i)
            if i >= 1:
                wait_stores(i - 1)
            if i + 1 < max_iters:
                start_load(i + 1)
            start_stores(i)
        wait_stores(max_iters - 1)

    return detile_kernel


@functools.lru_cache(maxsize=None)
def _build_gather():
    info = plsc.get_sparse_core_info()
    nw = info.num_cores * info.num_subcores
    b_per_w = BATCH // nw
    idx_rows = b_per_w // IDX_COLS
    mesh = plsc.VectorSubcoreMesh(core_axis_name="c", subcore_axis_name="s")

    @functools.partial(
        pl.kernel,
        out_type=jax.ShapeDtypeStruct((EMBED_DIM, BATCH), jnp.float32),
        mesh=mesh,
        scratch_types=[
            pltpu.VMEM((idx_rows, IDX_COLS), jnp.int32),
            pltpu.VMEM((idx_rows, IDX_COLS), jnp.int32),
            pltpu.VMEM((idx_rows, IDX_COLS), jnp.int32),
            pltpu.VMEM((idx_rows, IDX_COLS), jnp.int32),
            pltpu.VMEM((2, 1, b_per_w), jnp.int32),
            pltpu.VMEM((EMBED_DIM * EMBED_DIM // GRAN, GRAN),
                       jnp.float32),
            pltpu.VMEM((2, b_per_w, GRAN), jnp.float32),
            pltpu.VMEM((2, 1, b_per_w), jnp.float32),
            pltpu.SemaphoreType.DMA,
            pltpu.SemaphoreType.DMA,
            pltpu.SemaphoreType.DMA,
            pltpu.SemaphoreType.DMA,
        ],
        compiler_params=pltpu.CompilerParams(use_tc_tiling_on_sc=False,
                                             needs_layout_passes=False),
    )
    def gather_kernel(g_hbm, s_hbm, gs_hbm, sel_hbm, small_hbm, gran_hbm,
                      out_hbm, g_v, s_v, gs_v, sel_v, gc_v, small_v,
                      gath_v, comp_v, gsem_a, gsem_b, osem_a, osem_b):
        wid = lax.axis_index("s") * info.num_cores + lax.axis_index("c")
        pltpu.sync_copy(g_hbm.at[pl.ds(wid * idx_rows, idx_rows)], g_v)
        pltpu.sync_copy(s_hbm.at[pl.ds(wid * idx_rows, idx_rows)], s_v)
        pltpu.sync_copy(gs_hbm.at[pl.ds(wid * idx_rows, idx_rows)], gs_v)
        pltpu.sync_copy(sel_hbm.at[pl.ds(wid * idx_rows, idx_rows)], sel_v)
        pltpu.sync_copy(small_hbm, small_v)
        iota16 = lax.iota(jnp.int32, 16)
        gsems = (gsem_a, gsem_b)
        osems = (osem_a, osem_b)

        def fire(c_row, p):
            base = c_row * GPR
            for r in range(idx_rows):
                for q in range(IDX_COLS // 16):
                    gc_v[p, 0, pl.ds(r * IDX_COLS + q * 16, 16)] = (
                        g_v[r, pl.ds(q * 16, 16)] + base)
            pltpu.make_async_copy(
                gran_hbm.at[gc_v.at[p, 0]],
                gath_v.at[p],
                gsems[p]).start()

        def drain(p):
            pltpu.make_async_copy(
                gran_hbm.at[gc_v.at[p, 0]],
                gath_v.at[p],
                gsems[p]).wait()

        def compact_and_store(c_row, p):
            small_base = c_row * (EMBED_DIM // GRAN)
            for k in range(b_per_w // 16):
                rr, cc = (k * 16) // IDX_COLS, (k * 16) % IDX_COLS
                s_vec = s_v[rr, pl.ds(cc, 16)]
                rows_k = iota16 + (k * 16)
                main16 = plsc.load_gather(gath_v.at[p], [rows_k, s_vec])
                srow = gs_v[rr, pl.ds(cc, 16)] + small_base
                small16 = plsc.load_gather(small_v, [srow, s_vec])
                sel = sel_v[rr, pl.ds(cc, 16)]
                comp_v[p, 0, pl.ds(k * 16, 16)] = jnp.where(
                    sel > 0, small16, main16)
            pltpu.make_async_copy(
                comp_v.at[p],
                out_hbm.at[pl.ds(c_row, 1), pl.ds(wid * b_per_w, b_per_w)],
                osems[p]).start()

        def wait_out(p):
            pltpu.make_async_copy(
                comp_v.at[p],
                out_hbm.at[pl.ds(0, 1), pl.ds(0, b_per_w)],
                osems[p]).wait()

        fire(0, 0)

        def pair(p2, _):
            for par in range(2):
                c_row = p2 * 2 + par

                @pl.when(c_row + 1 < EMBED_DIM)
                def _():
                    fire(c_row + 1, 1 - par)

                drain(par)

                @pl.when(c_row >= 2)
                def _():
                    wait_out(par)

                compact_and_store(c_row, par)
            return 0

        lax.fori_loop(0, EMBED_DIM // 2, pair, 0)
        wait_out(0)
        wait_out(1)

    return gather_kernel


def kernel(indices, embed_cache):
    idx = indices.astype(jnp.int32)
    shape2d = (BATCH // IDX_COLS, IDX_COLS)
    g2d = (jnp.minimum(idx, VMAIN - 1) // GRAN).reshape(shape2d)
    s2d = (idx % GRAN).reshape(shape2d)
    gs2d = (jnp.clip(idx - VMAIN, 0, VOCAB - VMAIN - 1) // GRAN).reshape(
        shape2d)
    sel2d = (idx >= VMAIN).astype(jnp.int32).reshape(shape2d)
    small = embed_cache.T[:, VMAIN:].reshape(
        EMBED_DIM * (VOCAB - VMAIN) // GRAN, GRAN)
    flat = _build_detile()(embed_cache.T)
    out_t = _build_gather()(
        g2d, s2d, gs2d, sel2d, small,
        flat.reshape(VMAIN * EMBED_DIM // GRAN, GRAN))
    return out_t.T


# speedup vs baseline: 2.4397x; 1.0307x over previous
import functools

import jax
import jax.numpy as jnp
from jax import lax
from jax.experimental import pallas as pl
from jax.experimental.pallas import tpu as pltpu
from jax.experimental.pallas import tpu_sc as plsc

VOCAB = 1_000_000
EMBED_DIM = 64
BATCH = 16384
GRAN = 16
IDX_COLS = 128
VMAIN = 999_936
GPR = VMAIN // GRAN
SEG = 7936
N_SEGS = VMAIN // SEG


@functools.lru_cache(maxsize=None)
def _build_detile():
    info = plsc.get_sparse_core_info()
    nw = info.num_cores * info.num_subcores
    n_items = (EMBED_DIM // 8) * N_SEGS
    max_iters = (n_items + nw - 1) // nw
    mesh = plsc.VectorSubcoreMesh(core_axis_name="c", subcore_axis_name="s")

    @functools.partial(
        pl.kernel,
        out_type=jax.ShapeDtypeStruct((VMAIN * EMBED_DIM,), jnp.float32),
        mesh=mesh,
        scratch_types=[
            pltpu.VMEM((8, SEG), jnp.float32),
            pltpu.VMEM((8, SEG), jnp.float32),
            pltpu.SemaphoreType.DMA,
            pltpu.SemaphoreType.DMA,
            pltpu.SemaphoreType.DMA,
            pltpu.SemaphoreType.DMA,
        ],
        compiler_params=pltpu.CompilerParams(use_tc_tiling_on_sc=True,
                                             needs_layout_passes=False),
    )
    def detile_kernel(table_hbm, flat_hbm, buf_a, buf_b,
                      lsem_a, lsem_b, ssem_a, ssem_b):
        wid = lax.axis_index("s") * info.num_cores + lax.axis_index("c")
        bufs = (buf_a, buf_b)
        lsems, ssems = (lsem_a, lsem_b), (ssem_a, ssem_b)

        def valid(i):
            return wid + i * nw < n_items

        def band_seg(i):
            item = wid + i * nw
            return item // N_SEGS, item % N_SEGS

        def start_load(i):
            band, s = band_seg(i)
            p = i % 2

            @pl.when(valid(i))
            def _():
                pltpu.make_async_copy(
                    table_hbm.at[pl.ds(band * 8, 8), pl.ds(s * SEG, SEG)],
                    bufs[p], lsems[p]).start()

        def wait_load(i):
            p = i % 2

            @pl.when(valid(i))
            def _():
                pltpu.make_async_copy(
                    table_hbm.at[pl.ds(0, 8), pl.ds(0, SEG)],
                    bufs[p], lsems[p]).wait()

        def start_stores(i):
            band, s = band_seg(i)
            p = i % 2

            @pl.when(valid(i))
            def _():
                for r in range(8):
                    pltpu.make_async_copy(
                        bufs[p].at[r],
                        flat_hbm.at[pl.ds((band * 8 + r) * VMAIN + s * SEG,
                                          SEG)],
                        ssems[p]).start()

        def wait_stores(i):
            p = i % 2

            @pl.when(valid(i))
            def _():
                pltpu.make_async_copy(
                    table_hbm.at[pl.ds(0, 8), pl.ds(0, SEG)],
                    bufs[p], ssems[p]).wait()

        start_load(0)
        for i in range(max_iters):
            wait_load(i)
            if i >= 1:
                wait_stores(i - 1)
            if i + 1 < max_iters:
                start_load(i + 1)
            start_stores(i)
        wait_stores(max_iters - 1)

    return detile_kernel


@functools.lru_cache(maxsize=None)
def _build_gather():
    info = plsc.get_sparse_core_info()
    nw = info.num_cores * info.num_subcores
    b_per_w = BATCH // nw
    idx_rows = b_per_w // IDX_COLS
    mesh = plsc.VectorSubcoreMesh(core_axis_name="c", subcore_axis_name="s")

    pair_w = 2 * b_per_w
    n_pairs = EMBED_DIM // 2

    @functools.partial(
        pl.kernel,
        out_type=jax.ShapeDtypeStruct((EMBED_DIM, BATCH), jnp.float32),
        mesh=mesh,
        scratch_types=[
            pltpu.VMEM((idx_rows, IDX_COLS), jnp.int32),
            pltpu.VMEM((idx_rows, IDX_COLS), jnp.int32),
            pltpu.VMEM((idx_rows, IDX_COLS), jnp.int32),
            pltpu.VMEM((idx_rows, IDX_COLS), jnp.int32),
            pltpu.VMEM((idx_rows, IDX_COLS), jnp.int32),
            pltpu.VMEM((2, 1, pair_w), jnp.int32),
            pltpu.VMEM((EMBED_DIM * EMBED_DIM // GRAN, GRAN),
                       jnp.float32),
            pltpu.VMEM((2, pair_w, GRAN), jnp.float32),
            pltpu.VMEM((2, 1, pair_w), jnp.float32),
            pltpu.SemaphoreType.DMA,
            pltpu.SemaphoreType.DMA,
            pltpu.SemaphoreType.DMA,
            pltpu.SemaphoreType.DMA,
        ],
        compiler_params=pltpu.CompilerParams(use_tc_tiling_on_sc=False,
                                             needs_layout_passes=False),
    )
    def gather_kernel(idx_hbm, small_hbm, gran_hbm, out_hbm,
                      idx_v, g_v, s_v, gs_v, sel_v, gc_v, small_v,
                      gath_v, comp_v, gsem_a, gsem_b, osem_a, osem_b):
        wid = lax.axis_index("s") * info.num_cores + lax.axis_index("c")
        pltpu.sync_copy(idx_hbm.at[pl.ds(wid * idx_rows, idx_rows)], idx_v)
        pltpu.sync_copy(small_hbm, small_v)
        iota16 = lax.iota(jnp.int32, 16)
        gsems = (gsem_a, gsem_b)
        osems = (osem_a, osem_b)
        for r in range(idx_rows):
            for q in range(IDX_COLS // 16):
                sl = pl.ds(q * 16, 16)
                v = idx_v[r, sl]
                g_v[r, sl] = jnp.minimum(v, VMAIN - 1) // GRAN
                s_v[r, sl] = lax.rem(v, GRAN)
                gs_v[r, sl] = jnp.clip(v - VMAIN, 0,
                                       VOCAB - VMAIN - 1) // GRAN
                sel_v[r, sl] = jnp.where(v >= VMAIN, 1, 0)

        def fire(j, p):
            for half in range(2):
                base = (2 * j + half) * GPR
                for r in range(idx_rows):
                    for q in range(IDX_COLS // 16):
                        gc_v[p, 0, pl.ds(half * b_per_w + r * IDX_COLS
                                         + q * 16, 16)] = (
                            g_v[r, pl.ds(q * 16, 16)] + base)
            pltpu.make_async_copy(
                gran_hbm.at[gc_v.at[p, 0]],
                gath_v.at[p],
                gsems[p]).start()

        def drain(p):
            pltpu.make_async_copy(
                gran_hbm.at[gc_v.at[p, 0]],
                gath_v.at[p],
                gsems[p]).wait()

        def compact_and_store(j, p):
            for half in range(2):
                c_row = 2 * j + half
                small_base = c_row * (EMBED_DIM // GRAN)
                for k in range(b_per_w // 16):
                    rr, cc = (k * 16) // IDX_COLS, (k * 16) % IDX_COLS
                    s_vec = s_v[rr, pl.ds(cc, 16)]
                    rows_k = iota16 + (half * b_per_w + k * 16)
                    main16 = plsc.load_gather(gath_v.at[p], [rows_k, s_vec])
                    srow = gs_v[rr, pl.ds(cc, 16)] + small_base
                    small16 = plsc.load_gather(small_v, [srow, s_vec])
                    sel = sel_v[rr, pl.ds(cc, 16)]
                    comp_v[p, 0, pl.ds(half * b_per_w + k * 16, 16)] = (
                        jnp.where(sel > 0, small16, main16))
                pltpu.make_async_copy(
                    comp_v.at[p, :, pl.ds(half * b_per_w, b_per_w)],
                    out_hbm.at[pl.ds(c_row, 1),
                               pl.ds(wid * b_per_w, b_per_w)],
                    osems[p]).start()

        def wait_out(p):
            pltpu.make_async_copy(
                comp_v.at[p],
                out_hbm.at[pl.ds(0, 2), pl.ds(0, b_per_w)],
                osems[p]).wait()

        fire(0, 0)

        def pair2(p2, _):
            for par in range(2):
                j = p2 * 2 + par

                @pl.when(j + 1 < n_pairs)
                def _():
                    fire(j + 1, 1 - par)

                drain(par)

                @pl.when(j >= 2)
                def _():
                    wait_out(par)

                compact_and_store(j, par)
            return 0

        lax.fori_loop(0, n_pairs // 2, pair2, 0)
        wait_out(0)
        wait_out(1)

    return gather_kernel


def kernel(indices, embed_cache):
    idx2d = indices.astype(jnp.int32).reshape(BATCH // IDX_COLS, IDX_COLS)
    small = embed_cache.T[:, VMAIN:].reshape(
        EMBED_DIM * (VOCAB - VMAIN) // GRAN, GRAN)
    flat = _build_detile()(embed_cache.T)
    out_t = _build_gather()(
        idx2d, small, flat.reshape(VMAIN * EMBED_DIM // GRAN, GRAN))
    return out_t.T
